# baseline (speedup 1.0000x reference)
import functools

import jax
import jax.numpy as jnp
from jax import lax
from jax.experimental import pallas as pl
from jax.experimental.pallas import tpu as pltpu
from jax.experimental.pallas import tpu_sc as plsc

NEG_SLOPE = 0.2
EPS = 1e-16

NC = 2
NS = 16
L = 16

HW = 80


def _leaky(a):
    return jnp.where(a >= 0, a, NEG_SLOPE * a)


def _tail(bn):
    return (lax.broadcasted_iota(jnp.int32, (bn, 16), 1) == 0).astype(jnp.float32)


def _tc_pre1_body(x_ref, w_ref, asrc_ref, adst_ref, hpad_ref, aux_ref):
    h = jnp.dot(x_ref[...], w_ref[...], preferred_element_type=jnp.float32)
    bn = h.shape[0]
    asrc = jnp.sum(h * asrc_ref[0][None, :], axis=-1)
    adst = jnp.sum(h * adst_ref[0][None, :], axis=-1)
    wself = jnp.exp(_leaky(asrc + adst))
    tail = _tail(bn)
    hpad_ref[:, 0:64] = h[:, 0:64]
    hpad_ref[:, 64:80] = tail
    hpad_ref[:, 80:144] = h[:, 64:128]
    hpad_ref[:, 144:160] = tail
    aux_ref[0:1, :] = asrc[None, :]
    aux_ref[1:2, :] = adst[None, :]
    aux_ref[2:3, :] = asrc[None, :]
    aux_ref[3:4, :] = adst[None, :]
    aux_ref[4:5, :] = wself[None, :]
    aux_ref[5:6, :] = wself[None, :]
    aux_ref[6:7, :] = jnp.zeros((1, bn), jnp.float32)
    aux_ref[7:8, :] = jnp.zeros((1, bn), jnp.float32)


def _tc_pre1(x, w1, att_src1, att_dst1):
    n, _ = x.shape
    return pl.pallas_call(
        _tc_pre1_body,
        compiler_params=pltpu.CompilerParams(vmem_limit_bytes=100 * 2**20),
        out_shape=[
            jax.ShapeDtypeStruct((n, 2 * HW), jnp.float32),
            jax.ShapeDtypeStruct((8, n), jnp.float32),
        ],
    )(x, w1, att_src1, att_dst1)


def _tc_mid_body(a0_ref, a1_ref, hpad_ref, aux_ref, b1_ref, wmu_ref, wls_ref,
                 smu_ref, dmu_ref, sls_ref, dls_ref, hml_ref, aux2_ref):
    a0 = a0_ref[...]
    a1 = a1_ref[...]
    bn = a0.shape[0]
    wself = aux_ref[4:5, :].reshape(bn, 1)
    h1a = hpad_ref[:, 0:64]
    h1b = hpad_ref[:, 80:144]
    den0 = a0[:, 64:65] + wself + EPS
    den1 = a1[:, 64:65] + wself + EPS
    ga = (a0[:, 0:64] + wself * h1a) / den0 + b1_ref[0][None, 0:64]
    gb = (a1[:, 0:64] + wself * h1b) / den1 + b1_ref[0][None, 64:128]
    g = jnp.concatenate([ga, gb], axis=1)
    g = jnp.where(g > 0, g, jnp.exp(jnp.minimum(g, 0.0)) - 1.0)
    hmu = jnp.dot(g, wmu_ref[...], preferred_element_type=jnp.float32)
    hls = jnp.dot(g, wls_ref[...], preferred_element_type=jnp.float32)
    asrc_mu = jnp.sum(hmu * smu_ref[0][None, :], axis=-1)
    adst_mu = jnp.sum(hmu * dmu_ref[0][None, :], axis=-1)
    asrc_ls = jnp.sum(hls * sls_ref[0][None, :], axis=-1)
    adst_ls = jnp.sum(hls * dls_ref[0][None, :], axis=-1)
    wself_mu = jnp.exp(_leaky(asrc_mu + adst_mu))
    wself_ls = jnp.exp(_leaky(asrc_ls + adst_ls))
    tail = _tail(bn)
    hml_ref[:, 0:64] = hmu
    hml_ref[:, 64:80] = tail
    hml_ref[:, 80:144] = hls
    hml_ref[:, 144:160] = tail
    aux2_ref[0:1, :] = asrc_mu[None, :]
    aux2_ref[1:2, :] = adst_mu[None, :]
    aux2_ref[2:3, :] = asrc_ls[None, :]
    aux2_ref[3:4, :] = adst_ls[None, :]
    aux2_ref[4:5, :] = wself_mu[None, :]
    aux2_ref[5:6, :] = wself_ls[None, :]
    aux2_ref[6:7, :] = jnp.zeros((1, bn), jnp.float32)
    aux2_ref[7:8, :] = jnp.zeros((1, bn), jnp.float32)


def _tc_mid(a0, a1, hpad, aux, b1, wmu, wls, smu, dmu, sls, dls):
    n = hpad.shape[0]
    return pl.pallas_call(
        _tc_mid_body,
        compiler_params=pltpu.CompilerParams(vmem_limit_bytes=100 * 2**20),
        out_shape=[
            jax.ShapeDtypeStruct((n, 2 * HW), jnp.float32),
            jax.ShapeDtypeStruct((8, n), jnp.float32),
        ],
    )(a0, a1, hpad, aux, b1, wmu, wls, smu, dmu, sls, dls)


def _tc_post_body(a0_ref, a1_ref, hml_ref, aux2_ref, bmu_ref, bls_ref,
                  mu_ref, ls_ref):
    a0 = a0_ref[...]
    a1 = a1_ref[...]
    bn = a0.shape[0]
    hmu = hml_ref[:, 0:64]
    hls = hml_ref[:, 80:144]
    wmu = aux2_ref[4:5, :].reshape(bn, 1)
    wls = aux2_ref[5:6, :].reshape(bn, 1)
    mu_ref[...] = (a0[:, 0:64] + wmu * hmu) / (a0[:, 64:65] + wmu + EPS) \
        + bmu_ref[0][None, :]
    ls_ref[...] = (a1[:, 0:64] + wls * hls) / (a1[:, 64:65] + wls + EPS) \
        + bls_ref[0][None, :]


def _tc_post(a0, a1, hml, aux2, bmu, bls):
    n = hml.shape[0]
    return pl.pallas_call(
        _tc_post_body,
        compiler_params=pltpu.CompilerParams(vmem_limit_bytes=100 * 2**20),
        out_shape=[
            jax.ShapeDtypeStruct((n, 64), jnp.float32),
            jax.ShapeDtypeStruct((n, 64), jnp.float32),
        ],
    )(a0, a1, hml, aux2, bmu, bls)


def _sc_edge_pass(table2, esrc2, edst2, aux, zeros):
    n = aux.shape[1]
    np_ = zeros.shape[0]
    ch = esrc2.shape[1]
    e = esrc2.shape[0] * ch
    ept = e // NS
    npair = ept // (2 * ch)
    rpt = np_ // NS
    nvr = HW // L

    mesh = plsc.VectorSubcoreMesh(core_axis_name="c", subcore_axis_name="s")

    scratch = [
        pltpu.VMEM((n,), jnp.float32),
        pltpu.VMEM((n,), jnp.float32),
        pltpu.VMEM((2, ch), jnp.int32),
        pltpu.VMEM((2, ch), jnp.int32),
        pltpu.VMEM((2, ch), jnp.int32),
        pltpu.VMEM((ch,), jnp.float32),
        pltpu.VMEM((ch,), jnp.float32),
        pltpu.VMEM((ch, HW), jnp.float32),
        pltpu.VMEM((ch, HW), jnp.float32),
        pltpu.VMEM((ch, HW), jnp.float32),
        pltpu.VMEM((ch, HW), jnp.float32),
        pltpu.VMEM_SHARED((np_, HW), jnp.float32),
        pltpu.SemaphoreType.DMA,
        pltpu.SemaphoreType.DMA,
        pltpu.SemaphoreType.DMA,
        pltpu.SemaphoreType.DMA,
    ]

    @functools.partial(
        pl.kernel,
        out_type=jax.ShapeDtypeStruct((NC, np_, HW), jnp.float32),
        mesh=mesh,
        scratch_types=scratch,
        compiler_params=pltpu.CompilerParams(needs_layout_passes=False,
                                             use_tc_tiling_on_sc=False),
    )
    def body(table_h, esrc_h, edst_h, aux_h, zeros_h, out_h,
             asv, adv, sidxp, didxp, tidx, w0, w1, rows0, rows1,
             srows0, srows1, accum, gsem0, gsem1, ssem0, ssem1):
        c = lax.axis_index("c")
        s = lax.axis_index("s")

        pltpu.sync_copy(aux_h.at[2 * c], asv)
        pltpu.sync_copy(aux_h.at[2 * c + 1], adv)
        pltpu.sync_copy(zeros_h.at[pl.ds(s * rpt, rpt)],
                        accum.at[pl.ds(s * rpt, rpt)])
        plsc.subcore_barrier()

        rbase = s * (ept // ch)

        def scale(rows, srows, wv):
            def rscale(r, carry2):
                idxv = jnp.full((L,), r, jnp.int32)
                wb = plsc.load_gather(wv, [idxv])
                for j in range(nvr):
                    srows[r, pl.ds(j * L, L)] = rows[r, pl.ds(j * L, L)] * wb
                return carry2
            lax.fori_loop(0, ch, rscale, 0, unroll=4)

        def pair(g, carry):
            r0 = rbase + 2 * g
            pltpu.sync_copy(esrc_h.at[pl.ds(r0, 2)], sidxp)
            pltpu.sync_copy(edst_h.at[pl.ds(r0, 2)], didxp)
            for half in (0, 1):
                for j in range(ch // L):
                    tidx[half, pl.ds(j * L, L)] = \
                        sidxp[half, pl.ds(j * L, L)] * 2 + c
            gat0 = pltpu.async_copy(table_h.at[tidx.at[0]], rows0, gsem0)
            gat1 = pltpu.async_copy(table_h.at[tidx.at[1]], rows1, gsem1)
            for half, wv in ((0, w0), (1, w1)):
                for j in range(ch // L):
                    sv = sidxp[half, pl.ds(j * L, L)]
                    dv = didxp[half, pl.ds(j * L, L)]
                    a = plsc.load_gather(asv, [sv]) \
                        + plsc.load_gather(adv, [dv])
                    wv[pl.ds(j * L, L)] = jnp.exp(_leaky(a))
            gat0.wait()
            scale(rows0, srows0, w0)
            sc0 = pltpu.async_copy(srows0, accum.at[didxp.at[0]], ssem0,
                                   add=True)
            gat1.wait()
            scale(rows1, srows1, w1)
            sc1 = pltpu.async_copy(srows1, accum.at[didxp.at[1]], ssem1,
                                   add=True)
            sc0.wait()
            sc1.wait()
            return carry

        lax.fori_loop(0, npair, pair, 0)
        plsc.subcore_barrier()
        pltpu.sync_copy(accum.at[pl.ds(s * rpt, rpt)],
                        out_h.at[c, pl.ds(s * rpt, rpt)])

    return body(table2, esrc2, edst2, aux, zeros)


def kernel(x, edge_index, W1, att_src1, att_dst1, b1,
           W_mu, att_src_mu, att_dst_mu, b_mu,
           W_ls, att_src_ls, att_dst_ls, b_ls):
    n = x.shape[0]
    ei = edge_index.astype(jnp.int32)
    ch = 80
    esrc2 = ei[0].reshape(-1, ch)
    edst2 = ei[1].reshape(-1, ch)
    np_ = ((n + NS * 8 - 1) // (NS * 8)) * (NS * 8)
    zeros = jnp.zeros((np_, HW), jnp.float32)

    hpad, aux1 = _tc_pre1(x, W1, att_src1, att_dst1)
    acc1 = _sc_edge_pass(hpad.reshape(2 * n, HW), esrc2, edst2, aux1, zeros)
    hml, aux2 = _tc_mid(acc1[0, :n], acc1[1, :n], hpad, aux1, b1[None, :],
                        W_mu, W_ls, att_src_mu, att_dst_mu,
                        att_src_ls, att_dst_ls)
    acc2 = _sc_edge_pass(hml.reshape(2 * n, HW), esrc2, edst2, aux2, zeros)
    mu, ls = _tc_post(acc2[0, :n], acc2[1, :n], hml, aux2,
                      b_mu[None, :], b_ls[None, :])
    return (mu, ls)


# speedup vs baseline: 1.8989x; 1.8989x over previous
import functools

import jax
import jax.numpy as jnp
from jax import lax
from jax.experimental import pallas as pl
from jax.experimental.pallas import tpu as pltpu
from jax.experimental.pallas import tpu_sc as plsc

NEG_SLOPE = 0.2
EPS = 1e-16

NC = 2
NS = 16
L = 16

HW = 64


def _leaky(a):
    return jnp.where(a >= 0, a, NEG_SLOPE * a)


def _tail(bn):
    return (lax.broadcasted_iota(jnp.int32, (bn, 16), 1) == 0).astype(jnp.float32)


def _tc_pre1_body(x_ref, w_ref, asrc_ref, adst_ref, hpad_ref, aux_ref):
    h = jnp.dot(x_ref[...], w_ref[...], preferred_element_type=jnp.float32)
    bn = h.shape[0]
    asrc = jnp.sum(h * asrc_ref[0][None, :], axis=-1)
    adst = jnp.sum(h * adst_ref[0][None, :], axis=-1)
    wself = jnp.exp(_leaky(asrc + adst))
    hpad_ref[...] = h
    aux_ref[0:1, :] = asrc[None, :]
    aux_ref[1:2, :] = adst[None, :]
    aux_ref[2:3, :] = asrc[None, :]
    aux_ref[3:4, :] = adst[None, :]
    aux_ref[4:5, :] = wself[None, :]
    aux_ref[5:6, :] = wself[None, :]
    aux_ref[6:7, :] = jnp.zeros((1, bn), jnp.float32)
    aux_ref[7:8, :] = jnp.zeros((1, bn), jnp.float32)


def _tc_pre1(x, w1, att_src1, att_dst1):
    n, _ = x.shape
    return pl.pallas_call(
        _tc_pre1_body,
        compiler_params=pltpu.CompilerParams(vmem_limit_bytes=100 * 2**20),
        out_shape=[
            jax.ShapeDtypeStruct((n, 128), jnp.float32),
            jax.ShapeDtypeStruct((8, n), jnp.float32),
        ],
    )(x, w1, att_src1, att_dst1)


def _tc_mid_body(a0_ref, a1_ref, d0_ref, d1_ref, hpad_ref, aux_ref, b1_ref,
                 wmu_ref, wls_ref, smu_ref, dmu_ref, sls_ref, dls_ref,
                 hml_ref, aux2_ref):
    a0 = a0_ref[...]
    a1 = a1_ref[...]
    bn = a0.shape[0]
    wself = aux_ref[4:5, :].reshape(bn, 1)
    h1a = hpad_ref[:, 0:64]
    h1b = hpad_ref[:, 64:128]
    den0 = d0_ref[0:1, :].reshape(bn, 1) + wself + EPS
    den1 = d1_ref[0:1, :].reshape(bn, 1) + wself + EPS
    ga = (a0 + wself * h1a) / den0 + b1_ref[0][None, 0:64]
    gb = (a1 + wself * h1b) / den1 + b1_ref[0][None, 64:128]
    g = jnp.concatenate([ga, gb], axis=1)
    g = jnp.where(g > 0, g, jnp.exp(jnp.minimum(g, 0.0)) - 1.0)
    hmu = jnp.dot(g, wmu_ref[...], preferred_element_type=jnp.float32)
    hls = jnp.dot(g, wls_ref[...], preferred_element_type=jnp.float32)
    asrc_mu = jnp.sum(hmu * smu_ref[0][None, :], axis=-1)
    adst_mu = jnp.sum(hmu * dmu_ref[0][None, :], axis=-1)
    asrc_ls = jnp.sum(hls * sls_ref[0][None, :], axis=-1)
    adst_ls = jnp.sum(hls * dls_ref[0][None, :], axis=-1)
    wself_mu = jnp.exp(_leaky(asrc_mu + adst_mu))
    wself_ls = jnp.exp(_leaky(asrc_ls + adst_ls))
    hml_ref[:, 0:64] = hmu
    hml_ref[:, 64:128] = hls
    aux2_ref[0:1, :] = asrc_mu[None, :]
    aux2_ref[1:2, :] = adst_mu[None, :]
    aux2_ref[2:3, :] = asrc_ls[None, :]
    aux2_ref[3:4, :] = adst_ls[None, :]
    aux2_ref[4:5, :] = wself_mu[None, :]
    aux2_ref[5:6, :] = wself_ls[None, :]
    aux2_ref[6:7, :] = jnp.zeros((1, bn), jnp.float32)
    aux2_ref[7:8, :] = jnp.zeros((1, bn), jnp.float32)


def _tc_mid(a0, a1, d0, d1, hpad, aux, b1, wmu, wls, smu, dmu, sls, dls):
    n = hpad.shape[0]
    return pl.pallas_call(
        _tc_mid_body,
        compiler_params=pltpu.CompilerParams(vmem_limit_bytes=100 * 2**20),
        out_shape=[
            jax.ShapeDtypeStruct((n, 128), jnp.float32),
            jax.ShapeDtypeStruct((8, n), jnp.float32),
        ],
    )(a0, a1, d0, d1, hpad, aux, b1, wmu, wls, smu, dmu, sls, dls)


def _tc_post_body(a0_ref, a1_ref, d0_ref, d1_ref, hml_ref, aux2_ref,
                  bmu_ref, bls_ref, mu_ref, ls_ref):
    a0 = a0_ref[...]
    a1 = a1_ref[...]
    bn = a0.shape[0]
    hmu = hml_ref[:, 0:64]
    hls = hml_ref[:, 64:128]
    wmu = aux2_ref[4:5, :].reshape(bn, 1)
    wls = aux2_ref[5:6, :].reshape(bn, 1)
    dmu = d0_ref[0:1, :].reshape(bn, 1)
    dls = d1_ref[0:1, :].reshape(bn, 1)
    mu_ref[...] = (a0 + wmu * hmu) / (dmu + wmu + EPS) + bmu_ref[0][None, :]
    ls_ref[...] = (a1 + wls * hls) / (dls + wls + EPS) + bls_ref[0][None, :]


def _tc_post(a0, a1, d0, d1, hml, aux2, bmu, bls):
    n = hml.shape[0]
    return pl.pallas_call(
        _tc_post_body,
        compiler_params=pltpu.CompilerParams(vmem_limit_bytes=100 * 2**20),
        out_shape=[
            jax.ShapeDtypeStruct((n, 64), jnp.float32),
            jax.ShapeDtypeStruct((n, 64), jnp.float32),
        ],
    )(a0, a1, d0, d1, hml, aux2, bmu, bls)


def _sc_edge_pass(table2, esrc2, edst2, aux, zeros):
    n = aux.shape[1]
    np_ = zeros.shape[0]
    ch = esrc2.shape[1]
    e = esrc2.shape[0] * ch
    ept = e // NS
    npair = ept // (2 * ch)
    rpt = np_ // NS
    nvr = HW // L

    mesh = plsc.VectorSubcoreMesh(core_axis_name="c", subcore_axis_name="s")

    scratch = [
        pltpu.VMEM((n,), jnp.float32),
        pltpu.VMEM((n,), jnp.float32),
        pltpu.VMEM((2, ch), jnp.int32),
        pltpu.VMEM((2, ch), jnp.int32),
        pltpu.VMEM((2, ch), jnp.int32),
        pltpu.VMEM((ch,), jnp.float32),
        pltpu.VMEM((ch,), jnp.float32),
        pltpu.VMEM((np_,), jnp.float32),
        pltpu.VMEM((NS, rpt), jnp.float32),
        pltpu.VMEM((rpt,), jnp.float32),
        pltpu.VMEM((ch, HW), jnp.float32),
        pltpu.VMEM((ch, HW), jnp.float32),
        pltpu.VMEM_SHARED((np_, HW), jnp.float32),
        pltpu.VMEM_SHARED((NS, NS, rpt), jnp.float32),
        pltpu.SemaphoreType.DMA,
        pltpu.SemaphoreType.DMA,
        pltpu.SemaphoreType.DMA,
        pltpu.SemaphoreType.DMA,
    ]

    @functools.partial(
        pl.kernel,
        out_type=(jax.ShapeDtypeStruct((NC, np_, HW), jnp.float32),
                  jax.ShapeDtypeStruct((NC, np_), jnp.float32)),
        mesh=mesh,
        scratch_types=scratch,
        compiler_params=pltpu.CompilerParams(needs_layout_passes=False,
                                             use_tc_tiling_on_sc=False),
    )
    def body(table_h, esrc_h, edst_h, aux_h, zeros_h, out_h, den_h,
             asv, adv, sidxp, didxp, tidx, w0, w1, denloc, dbuf, dsum,
             rows0, rows1, accum, dstage, gsem0, gsem1, ssem0, ssem1):
        c = lax.axis_index("c")
        s = lax.axis_index("s")

        pltpu.sync_copy(aux_h.at[2 * c], asv)
        pltpu.sync_copy(aux_h.at[2 * c + 1], adv)
        pltpu.sync_copy(zeros_h.at[pl.ds(s * rpt, rpt)],
                        accum.at[pl.ds(s * rpt, rpt)])

        def zden(j, carry2):
            denloc[pl.ds(j * L, L)] = jnp.zeros((L,), jnp.float32)
            return carry2
        lax.fori_loop(0, np_ // L, zden, 0)
        plsc.subcore_barrier()

        rbase = s * (ept // ch)

        def scale(rows, wv):
            def rscale(i, carry2):
                r = 2 * i
                wba = plsc.load_gather(wv, [jnp.full((L,), r, jnp.int32)])
                wbb = plsc.load_gather(wv, [jnp.full((L,), r + 1, jnp.int32)])
                la = [rows[r, pl.ds(j * L, L)] for j in range(nvr)]
                lb = [rows[r + 1, pl.ds(j * L, L)] for j in range(nvr)]
                for j in range(nvr):
                    rows[r, pl.ds(j * L, L)] = la[j] * wba
                for j in range(nvr):
                    rows[r + 1, pl.ds(j * L, L)] = lb[j] * wbb
                return carry2
            lax.fori_loop(0, ch // 2, rscale, 0)

        def pair(g, carry):
            r0 = rbase + 2 * g
            pltpu.sync_copy(esrc_h.at[pl.ds(r0, 2)], sidxp)
            pltpu.sync_copy(edst_h.at[pl.ds(r0, 2)], didxp)
            for half in (0, 1):
                for j in range(ch // L):
                    tidx[half, pl.ds(j * L, L)] = \
                        sidxp[half, pl.ds(j * L, L)] * 2 + c
            gat0 = pltpu.async_copy(table_h.at[tidx.at[0]], rows0, gsem0)
            gat1 = pltpu.async_copy(table_h.at[tidx.at[1]], rows1, gsem1)
            for half, wv in ((0, w0), (1, w1)):
                for j in range(ch // L):
                    sv = sidxp[half, pl.ds(j * L, L)]
                    dv = didxp[half, pl.ds(j * L, L)]
                    a = plsc.load_gather(asv, [sv]) \
                        + plsc.load_gather(adv, [dv])
                    w = jnp.exp(_leaky(a))
                    wv[pl.ds(j * L, L)] = w
                    plsc.addupdate_scatter(denloc, [dv], w)
            gat0.wait()
            scale(rows0, w0)
            sc0 = pltpu.async_copy(rows0, accum.at[didxp.at[0]], ssem0,
                                   add=True)
            gat1.wait()
            scale(rows1, w1)
            sc1 = pltpu.async_copy(rows1, accum.at[didxp.at[1]], ssem1,
                                   add=True)
            sc0.wait()
            sc1.wait()
            return carry

        lax.fori_loop(0, npair, pair, 0)
        for k in range(NS):
            pltpu.sync_copy(denloc.at[pl.ds(k * rpt, rpt)], dstage.at[k, s])
        plsc.subcore_barrier()
        pltpu.sync_copy(accum.at[pl.ds(s * rpt, rpt)],
                        out_h.at[c, pl.ds(s * rpt, rpt)])
        pltpu.sync_copy(dstage.at[s], dbuf)

        def dred(v, carry2):
            acc16 = dbuf[0, pl.ds(v * L, L)]
            for t in range(1, NS):
                acc16 = acc16 + dbuf[t, pl.ds(v * L, L)]
            dsum[pl.ds(v * L, L)] = acc16
            return carry2
        lax.fori_loop(0, rpt // L, dred, 0)
        pltpu.sync_copy(dsum, den_h.at[c, pl.ds(s * rpt, rpt)])

    return body(table2, esrc2, edst2, aux, zeros)


def kernel(x, edge_index, W1, att_src1, att_dst1, b1,
           W_mu, att_src_mu, att_dst_mu, b_mu,
           W_ls, att_src_ls, att_dst_ls, b_ls):
    n = x.shape[0]
    ei = edge_index.astype(jnp.int32)
    ch = 80
    esrc2 = ei[0].reshape(-1, ch)
    edst2 = ei[1].reshape(-1, ch)
    np_ = ((n + NS * L - 1) // (NS * L)) * (NS * L)
    zeros = jnp.zeros((np_, HW), jnp.float32)

    hpad, aux1 = _tc_pre1(x, W1, att_src1, att_dst1)
    acc1, den1 = _sc_edge_pass(hpad.reshape(2 * n, HW), esrc2, edst2,
                               aux1, zeros)
    hml, aux2 = _tc_mid(acc1[0, :n], acc1[1, :n],
                        den1[0][None, :n], den1[1][None, :n],
                        hpad, aux1, b1[None, :],
                        W_mu, W_ls, att_src_mu, att_dst_mu,
                        att_src_ls, att_dst_ls)
    acc2, den2 = _sc_edge_pass(hml.reshape(2 * n, HW), esrc2, edst2,
                               aux2, zeros)
    mu, ls = _tc_post(acc2[0, :n], acc2[1, :n],
                      den2[0][None, :n], den2[1][None, :n],
                      hml, aux2, b_mu[None, :], b_ls[None, :])
    return (mu, ls)


# speedup vs baseline: 2.0885x; 1.0998x over previous
import functools

import jax
import jax.numpy as jnp
from jax import lax
from jax.experimental import pallas as pl
from jax.experimental.pallas import tpu as pltpu
from jax.experimental.pallas import tpu_sc as plsc

NEG_SLOPE = 0.2
EPS = 1e-16

NC = 2
NS = 16
L = 16

HW = 64


def _leaky(a):
    return jnp.where(a >= 0, a, NEG_SLOPE * a)


def _tail(bn):
    return (lax.broadcasted_iota(jnp.int32, (bn, 16), 1) == 0).astype(jnp.float32)


def _tc_pre1_body(x_ref, w_ref, asrc_ref, adst_ref, hpad_ref, aux_ref):
    h = jnp.dot(x_ref[...], w_ref[...], preferred_element_type=jnp.float32)
    bn = h.shape[0]
    asrc = jnp.sum(h * asrc_ref[0][None, :], axis=-1)
    adst = jnp.sum(h * adst_ref[0][None, :], axis=-1)
    wself = jnp.exp(_leaky(asrc + adst))
    hpad_ref[...] = h
    aux_ref[0:1, :] = asrc[None, :]
    aux_ref[1:2, :] = adst[None, :]
    aux_ref[2:3, :] = asrc[None, :]
    aux_ref[3:4, :] = adst[None, :]
    aux_ref[4:5, :] = wself[None, :]
    aux_ref[5:6, :] = wself[None, :]
    aux_ref[6:7, :] = jnp.zeros((1, bn), jnp.float32)
    aux_ref[7:8, :] = jnp.zeros((1, bn), jnp.float32)


def _tc_pre1(x, w1, att_src1, att_dst1):
    n, _ = x.shape
    return pl.pallas_call(
        _tc_pre1_body,
        compiler_params=pltpu.CompilerParams(vmem_limit_bytes=100 * 2**20),
        out_shape=[
            jax.ShapeDtypeStruct((n, 128), jnp.float32),
            jax.ShapeDtypeStruct((8, n), jnp.float32),
        ],
    )(x, w1, att_src1, att_dst1)


def _tc_mid_body(a0_ref, a1_ref, d0_ref, d1_ref, hpad_ref, aux_ref, b1_ref,
                 wmu_ref, wls_ref, smu_ref, dmu_ref, sls_ref, dls_ref,
                 hml_ref, aux2_ref):
    a0 = a0_ref[...]
    a1 = a1_ref[...]
    bn = a0.shape[0]
    wself = aux_ref[4:5, :].reshape(bn, 1)
    h1a = hpad_ref[:, 0:64]
    h1b = hpad_ref[:, 64:128]
    den0 = d0_ref[0:1, :].reshape(bn, 1) + wself + EPS
    den1 = d1_ref[0:1, :].reshape(bn, 1) + wself + EPS
    ga = (a0 + wself * h1a) / den0 + b1_ref[0][None, 0:64]
    gb = (a1 + wself * h1b) / den1 + b1_ref[0][None, 64:128]
    g = jnp.concatenate([ga, gb], axis=1)
    g = jnp.where(g > 0, g, jnp.exp(jnp.minimum(g, 0.0)) - 1.0)
    hmu = jnp.dot(g, wmu_ref[...], preferred_element_type=jnp.float32)
    hls = jnp.dot(g, wls_ref[...], preferred_element_type=jnp.float32)
    asrc_mu = jnp.sum(hmu * smu_ref[0][None, :], axis=-1)
    adst_mu = jnp.sum(hmu * dmu_ref[0][None, :], axis=-1)
    asrc_ls = jnp.sum(hls * sls_ref[0][None, :], axis=-1)
    adst_ls = jnp.sum(hls * dls_ref[0][None, :], axis=-1)
    wself_mu = jnp.exp(_leaky(asrc_mu + adst_mu))
    wself_ls = jnp.exp(_leaky(asrc_ls + adst_ls))
    hml_ref[:, 0:64] = hmu
    hml_ref[:, 64:128] = hls
    aux2_ref[0:1, :] = asrc_mu[None, :]
    aux2_ref[1:2, :] = adst_mu[None, :]
    aux2_ref[2:3, :] = asrc_ls[None, :]
    aux2_ref[3:4, :] = adst_ls[None, :]
    aux2_ref[4:5, :] = wself_mu[None, :]
    aux2_ref[5:6, :] = wself_ls[None, :]
    aux2_ref[6:7, :] = jnp.zeros((1, bn), jnp.float32)
    aux2_ref[7:8, :] = jnp.zeros((1, bn), jnp.float32)


def _tc_mid(a0, a1, d0, d1, hpad, aux, b1, wmu, wls, smu, dmu, sls, dls):
    n = hpad.shape[0]
    return pl.pallas_call(
        _tc_mid_body,
        compiler_params=pltpu.CompilerParams(vmem_limit_bytes=100 * 2**20),
        out_shape=[
            jax.ShapeDtypeStruct((n, 128), jnp.float32),
            jax.ShapeDtypeStruct((8, n), jnp.float32),
        ],
    )(a0, a1, d0, d1, hpad, aux, b1, wmu, wls, smu, dmu, sls, dls)


def _tc_post_body(a0_ref, a1_ref, d0_ref, d1_ref, hml_ref, aux2_ref,
                  bmu_ref, bls_ref, mu_ref, ls_ref):
    a0 = a0_ref[...]
    a1 = a1_ref[...]
    bn = a0.shape[0]
    hmu = hml_ref[:, 0:64]
    hls = hml_ref[:, 64:128]
    wmu = aux2_ref[4:5, :].reshape(bn, 1)
    wls = aux2_ref[5:6, :].reshape(bn, 1)
    dmu = d0_ref[0:1, :].reshape(bn, 1)
    dls = d1_ref[0:1, :].reshape(bn, 1)
    mu_ref[...] = (a0 + wmu * hmu) / (dmu + wmu + EPS) + bmu_ref[0][None, :]
    ls_ref[...] = (a1 + wls * hls) / (dls + wls + EPS) + bls_ref[0][None, :]


def _tc_post(a0, a1, d0, d1, hml, aux2, bmu, bls):
    n = hml.shape[0]
    return pl.pallas_call(
        _tc_post_body,
        compiler_params=pltpu.CompilerParams(vmem_limit_bytes=100 * 2**20),
        out_shape=[
            jax.ShapeDtypeStruct((n, 64), jnp.float32),
            jax.ShapeDtypeStruct((n, 64), jnp.float32),
        ],
    )(a0, a1, d0, d1, hml, aux2, bmu, bls)


def _sc_edge_pass(table2, eidx, aux, zeros):
    n = aux.shape[1]
    np_ = zeros.shape[0]
    ch = eidx.shape[3]
    e = eidx.shape[0] * 2 * ch
    ept = e // NS
    npair = ept // (2 * ch)
    rpt = np_ // NS
    nvr = HW // L

    mesh = plsc.VectorSubcoreMesh(core_axis_name="c", subcore_axis_name="s")

    scratch = [
        pltpu.VMEM((n,), jnp.float32),
        pltpu.VMEM((n,), jnp.float32),
        pltpu.VMEM((2, 2, ch), jnp.int32),
        pltpu.VMEM((2, ch), jnp.int32),
        pltpu.VMEM((ch,), jnp.float32),
        pltpu.VMEM((ch,), jnp.float32),
        pltpu.VMEM((np_,), jnp.float32),
        pltpu.VMEM((NS, rpt), jnp.float32),
        pltpu.VMEM((rpt,), jnp.float32),
        pltpu.VMEM((ch, HW), jnp.float32),
        pltpu.VMEM((ch, HW), jnp.float32),
        pltpu.VMEM_SHARED((np_, HW), jnp.float32),
        pltpu.VMEM_SHARED((NS, NS, rpt), jnp.float32),
        pltpu.SemaphoreType.DMA,
        pltpu.SemaphoreType.DMA,
        pltpu.SemaphoreType.DMA,
        pltpu.SemaphoreType.DMA,
    ]

    @functools.partial(
        pl.kernel,
        out_type=(jax.ShapeDtypeStruct((NC, np_, HW), jnp.float32),
                  jax.ShapeDtypeStruct((NC, np_), jnp.float32)),
        mesh=mesh,
        scratch_types=scratch,
        compiler_params=pltpu.CompilerParams(needs_layout_passes=False,
                                             use_tc_tiling_on_sc=False),
    )
    def body(table_h, eidx_h, aux_h, zeros_h, out_h, den_h,
             asv, adv, idxb, tidx, w0, w1, denloc, dbuf, dsum,
             rows0, rows1, accum, dstage, gsem0, gsem1, ssem0, ssem1):
        c = lax.axis_index("c")
        s = lax.axis_index("s")

        pltpu.sync_copy(aux_h.at[2 * c], asv)
        pltpu.sync_copy(aux_h.at[2 * c + 1], adv)
        pltpu.sync_copy(zeros_h.at[pl.ds(s * rpt, rpt)],
                        accum.at[pl.ds(s * rpt, rpt)])

        def zden(j, carry2):
            denloc[pl.ds(j * L, L)] = jnp.zeros((L,), jnp.float32)
            return carry2
        lax.fori_loop(0, np_ // L, zden, 0)
        plsc.subcore_barrier()

        rbase = s * npair

        def scale(rows, wv):
            def rscale(i, carry2):
                r = 2 * i
                wba = plsc.load_gather(wv, [jnp.full((L,), r, jnp.int32)])
                wbb = plsc.load_gather(wv, [jnp.full((L,), r + 1, jnp.int32)])
                la = [rows[r, pl.ds(j * L, L)] for j in range(nvr)]
                lb = [rows[r + 1, pl.ds(j * L, L)] for j in range(nvr)]
                for j in range(nvr):
                    rows[r, pl.ds(j * L, L)] = la[j] * wba
                for j in range(nvr):
                    rows[r + 1, pl.ds(j * L, L)] = lb[j] * wbb
                return carry2
            lax.fori_loop(0, ch // 2, rscale, 0)

        def pair(g, carry):
            pltpu.sync_copy(eidx_h.at[rbase + g], idxb)
            for half in (0, 1):
                for j in range(ch // L):
                    tidx[half, pl.ds(j * L, L)] = \
                        idxb[0, half, pl.ds(j * L, L)] * 2 + c
            gat0 = pltpu.async_copy(table_h.at[tidx.at[0]], rows0, gsem0)
            gat1 = pltpu.async_copy(table_h.at[tidx.at[1]], rows1, gsem1)
            for half, wv in ((0, w0), (1, w1)):
                for j in range(ch // L):
                    sv = idxb[0, half, pl.ds(j * L, L)]
                    dv = idxb[1, half, pl.ds(j * L, L)]
                    a = plsc.load_gather(asv, [sv]) \
                        + plsc.load_gather(adv, [dv])
                    w = jnp.exp(_leaky(a))
                    wv[pl.ds(j * L, L)] = w
                    plsc.addupdate_scatter(denloc, [dv], w)
            gat0.wait()
            scale(rows0, w0)
            sc0 = pltpu.async_copy(rows0, accum.at[idxb.at[1, 0]], ssem0,
                                   add=True)
            gat1.wait()
            scale(rows1, w1)
            sc1 = pltpu.async_copy(rows1, accum.at[idxb.at[1, 1]], ssem1,
                                   add=True)
            sc0.wait()
            sc1.wait()
            return carry

        lax.fori_loop(0, npair, pair, 0)
        for k in range(NS):
            pltpu.sync_copy(denloc.at[pl.ds(k * rpt, rpt)], dstage.at[k, s])
        plsc.subcore_barrier()
        pltpu.sync_copy(accum.at[pl.ds(s * rpt, rpt)],
                        out_h.at[c, pl.ds(s * rpt, rpt)])
        pltpu.sync_copy(dstage.at[s], dbuf)

        def dred(v, carry2):
            acc16 = dbuf[0, pl.ds(v * L, L)]
            for t in range(1, NS):
                acc16 = acc16 + dbuf[t, pl.ds(v * L, L)]
            dsum[pl.ds(v * L, L)] = acc16
            return carry2
        lax.fori_loop(0, rpt // L, dred, 0)
        pltpu.sync_copy(dsum, den_h.at[c, pl.ds(s * rpt, rpt)])

    return body(table2, eidx, aux, zeros)


def kernel(x, edge_index, W1, att_src1, att_dst1, b1,
           W_mu, att_src_mu, att_dst_mu, b_mu,
           W_ls, att_src_ls, att_dst_ls, b_ls):
    n = x.shape[0]
    ei = edge_index.astype(jnp.int32)
    ch = 80
    eidx = jnp.stack([ei[0].reshape(-1, 2, ch), ei[1].reshape(-1, 2, ch)],
                     axis=1)
    np_ = ((n + NS * L - 1) // (NS * L)) * (NS * L)
    zeros = jnp.zeros((np_, HW), jnp.float32)

    hpad, aux1 = _tc_pre1(x, W1, att_src1, att_dst1)
    acc1, den1 = _sc_edge_pass(hpad.reshape(2 * n, HW), eidx, aux1, zeros)
    hml, aux2 = _tc_mid(acc1[0, :n], acc1[1, :n],
                        den1[0][None, :n], den1[1][None, :n],
                        hpad, aux1, b1[None, :],
                        W_mu, W_ls, att_src_mu, att_dst_mu,
                        att_src_ls, att_dst_ls)
    acc2, den2 = _sc_edge_pass(hml.reshape(2 * n, HW), eidx, aux2, zeros)
    mu, ls = _tc_post(acc2[0, :n], acc2[1, :n],
                      den2[0][None, :n], den2[1][None, :n],
                      hml, aux2, b_mu[None, :], b_ls[None, :])
    return (mu, ls)
